# baseline (speedup 1.0000x reference)
import functools

import jax
import jax.numpy as jnp
from jax import lax
from jax.experimental import pallas as pl
from jax.experimental.pallas import tpu as pltpu
from jax.experimental.pallas import tpu_sc as plsc

NUM_WIFI = 100000
E = 128
T = 50
B = 4096
NW = 32
ROWS_PER_W = B // NW
CB = 4
NCHUNK = ROWS_PER_W // CB
NV = E // 16
OROW = T + 1
OWORDS = CB * OROW * E


def _sc_body(rssi_hbm, bssid_hbm, table_hbm, pos_hbm, cls_hbm, out_hbm,
             idx_all, rssi_all, gbuf_a, gbuf_b, obuf_a, obuf_b,
             posc, clsv, gsem_a, gsem_b, osem_a, osem_b):
    wid = lax.axis_index("s") * 2 + lax.axis_index("c")
    wbase = wid * ROWS_PER_W

    pltpu.sync_copy(bssid_hbm.at[pl.ds(wbase * T, ROWS_PER_W * T)], idx_all)
    pltpu.sync_copy(rssi_hbm.at[pl.ds(wbase * T, ROWS_PER_W * T)],
                    rssi_all.at[pl.ds(0, ROWS_PER_W * T)])

    pltpu.sync_copy(pos_hbm.at[pl.ds(0, 56), :], posc)
    pltpu.sync_copy(cls_hbm, clsv)
    for j in range(NV):
        s = pl.ds(j * 16, 16)
        posc[0, s] = posc[0, s] + clsv[s]
    for obuf in (obuf_a, obuf_b):
        for bi in range(CB):
            for j in range(NV):
                obuf[pl.ds(bi * OROW * E + j * 16, 16)] = posc[0, pl.ds(j * 16, 16)]

    def out_region(c):
        return out_hbm.at[pl.ds((wbase + c * CB) * OROW * E, OWORDS)]

    def idx_slice(c):
        return idx_all.at[pl.ds(c * CB * T, CB * T)]

    def prefetch(c, gbuf, gsem):
        pltpu.async_copy(table_hbm.at[idx_slice(c)], gbuf, gsem)

    HW = CB // 2 * OROW * E

    def half_region(c, h):
        return out_hbm.at[pl.ds((wbase + c * CB) * OROW * E + h * HW, HW)]

    def compute(c, gbuf, gsem, obuf, osem):
        pltpu.make_async_copy(table_hbm.at[idx_slice(c)], gbuf, gsem).wait()
        for h in range(2):
            pltpu.make_async_copy(
                obuf.at[pl.ds(h * HW, HW)], half_region(c, h), osem).wait()

        for h in range(2):
            def t_body(t, carry):
                pcs = [posc[t + 1, pl.ds(j * 16, 16)] for j in range(NV)]
                for bi in (2 * h, 2 * h + 1):
                    rv = rssi_all[pl.ds(c * CB * T + bi * T + t, 16)]
                    bc = jnp.full((16,), rv[0], dtype=jnp.float32)
                    rg = bi * T + t
                    ob = (bi * OROW + 1 + t) * E
                    for j in range(NV):
                        obuf[pl.ds(ob + j * 16, 16)] = (
                            gbuf[rg, pl.ds(j * 16, 16)] + (pcs[j] + bc))
                return carry

            lax.fori_loop(0, T, t_body, 0)
            pltpu.async_copy(obuf.at[pl.ds(h * HW, HW)], half_region(c, h), osem)

    prefetch(0, gbuf_a, gsem_a)
    for h in range(2):
        pltpu.async_copy(obuf_a.at[pl.ds(h * HW, HW)], half_region(0, h), osem_a)
        pltpu.async_copy(obuf_b.at[pl.ds(h * HW, HW)], half_region(1, h), osem_b)

    def chunk_pair(c2, carry):
        c = 2 * c2
        prefetch(c + 1, gbuf_b, gsem_b)
        compute(c, gbuf_a, gsem_a, obuf_a, osem_a)

        @pl.when(c2 < NCHUNK // 2 - 1)
        def _():
            prefetch(c + 2, gbuf_a, gsem_a)

        compute(c + 1, gbuf_b, gsem_b, obuf_b, osem_b)
        return carry

    lax.fori_loop(0, NCHUNK // 2, chunk_pair, 0)

    for h in range(2):
        pltpu.make_async_copy(obuf_a.at[pl.ds(h * HW, HW)],
                              half_region(NCHUNK - 2, h), osem_a).wait()
        pltpu.make_async_copy(obuf_b.at[pl.ds(h * HW, HW)],
                              half_region(NCHUNK - 1, h), osem_b).wait()


@jax.jit
def _anchor2token(rssi_f, bssid_f, table, pos, cls_f):
    mesh = plsc.VectorSubcoreMesh(core_axis_name="c", subcore_axis_name="s")
    k = functools.partial(
        pl.kernel,
        mesh=mesh,
        compiler_params=pltpu.CompilerParams(use_tc_tiling_on_sc=True),
        out_type=jax.ShapeDtypeStruct((B * OROW * E,), jnp.float32),
        scratch_types=[
            pltpu.VMEM((ROWS_PER_W * T,), jnp.int32),
            pltpu.VMEM((ROWS_PER_W * T + 16,), jnp.float32),
            pltpu.VMEM((CB * T, E), jnp.float32),
            pltpu.VMEM((CB * T, E), jnp.float32),
            pltpu.VMEM((OWORDS,), jnp.float32),
            pltpu.VMEM((OWORDS,), jnp.float32),
            pltpu.VMEM((56, E), jnp.float32),
            pltpu.VMEM((E,), jnp.float32),
            pltpu.SemaphoreType.DMA,
            pltpu.SemaphoreType.DMA,
            pltpu.SemaphoreType.DMA,
            pltpu.SemaphoreType.DMA,
        ],
    )(_sc_body)
    return k(rssi_f, bssid_f, table, pos, cls_f)


def kernel(rssi, bssid, bssid_table, pos_table, cls_token):
    rssi_f = rssi.reshape(B * T)
    bssid_f = bssid.reshape(B * T).astype(jnp.int32)
    cls_f = cls_token.reshape(E)
    out = _anchor2token(rssi_f, bssid_f, bssid_table, pos_table, cls_f)
    return out.reshape(B, T + 1, E)


# speedup vs baseline: 2.8313x; 2.8313x over previous
import functools

import jax
import jax.numpy as jnp
from jax import lax
from jax.experimental import pallas as pl
from jax.experimental.pallas import tpu as pltpu
from jax.experimental.pallas import tpu_sc as plsc

NUM_WIFI = 100000
E = 128
T = 50
B = 4096
NW = 32
RW = B // NW
NV = E // 16
OROW = T + 1


def _sc_body(rssi_hbm, bssid_hbm, table_hbm, pos_hbm, cls_hbm, out_hbm,
             idx_t, rssi_t, gbuf_a, gbuf_b, obuf_a, obuf_b, posc, clsv,
             gsem_a, gsem_b, osem_a, osem_b):
    wid = lax.axis_index("s") * 2 + lax.axis_index("c")
    wbase = wid * RW

    pltpu.sync_copy(bssid_hbm.at[:, pl.ds(wbase, RW)], idx_t)
    pltpu.sync_copy(rssi_hbm.at[:, pl.ds(wbase, RW)], rssi_t)

    pltpu.sync_copy(pos_hbm.at[pl.ds(0, OROW * E)], posc)
    pltpu.sync_copy(cls_hbm, clsv)
    for j in range(NV):
        s = pl.ds(j * 16, 16)
        posc[s] = posc[s] + clsv[s]

    def out_plane(p):
        return out_hbm.at[pl.ds(p * B + wbase, RW), :]

    def fill_row(r, carry):
        for j in range(NV):
            obuf_a[r, pl.ds(j * 16, 16)] = posc[pl.ds(j * 16, 16)]
        return carry

    lax.fori_loop(0, RW, fill_row, 0)
    pltpu.sync_copy(obuf_a, out_plane(0))

    def prefetch(t, gbuf, gsem):
        pltpu.async_copy(table_hbm.at[idx_t.at[t]], gbuf, gsem)

    def compute(t, gbuf, gsem, obuf, osem):
        pltpu.make_async_copy(table_hbm.at[idx_t.at[t]], gbuf, gsem).wait()
        pltpu.make_async_copy(obuf, out_plane(t + 1), osem).wait()
        pcs = [posc[pl.ds((t + 1) * E + j * 16, 16)] for j in range(NV)]

        def i16_body(i, carry):
            rv = rssi_t[t, pl.ds(i * 16, 16)]
            for l in range(16):
                bc = jnp.full((16,), rv[l], dtype=jnp.float32)
                for j in range(NV):
                    s = pl.ds(j * 16, 16)
                    obuf[i * 16 + l, s] = gbuf[i * 16 + l, s] + (pcs[j] + bc)
            return carry

        lax.fori_loop(0, RW // 16, i16_body, 0)
        pltpu.async_copy(obuf, out_plane(t + 1), osem)

    prefetch(0, gbuf_a, gsem_a)
    pltpu.async_copy(obuf_a, out_plane(1), osem_a)
    pltpu.async_copy(obuf_b, out_plane(2), osem_b)

    def plane_pair(t2, carry):
        t = 2 * t2
        prefetch(t + 1, gbuf_b, gsem_b)
        compute(t, gbuf_a, gsem_a, obuf_a, osem_a)

        @pl.when(t2 < T // 2 - 1)
        def _():
            prefetch(t + 2, gbuf_a, gsem_a)

        compute(t + 1, gbuf_b, gsem_b, obuf_b, osem_b)
        return carry

    lax.fori_loop(0, T // 2, plane_pair, 0)

    pltpu.make_async_copy(obuf_a, out_plane(T - 1), osem_a).wait()
    pltpu.make_async_copy(obuf_b, out_plane(T), osem_b).wait()


@jax.jit
def _anchor2token(rssi_tm, bssid_tm, table, pos_f, cls_f):
    mesh = plsc.VectorSubcoreMesh(core_axis_name="c", subcore_axis_name="s")
    k = functools.partial(
        pl.kernel,
        mesh=mesh,
        out_type=jax.ShapeDtypeStruct((OROW * B, E), jnp.float32),
        scratch_types=[
            pltpu.VMEM((T, RW), jnp.int32),
            pltpu.VMEM((T, RW), jnp.float32),
            pltpu.VMEM((RW, E), jnp.float32),
            pltpu.VMEM((RW, E), jnp.float32),
            pltpu.VMEM((RW, E), jnp.float32),
            pltpu.VMEM((RW, E), jnp.float32),
            pltpu.VMEM((OROW * E,), jnp.float32),
            pltpu.VMEM((E,), jnp.float32),
            pltpu.SemaphoreType.DMA,
            pltpu.SemaphoreType.DMA,
            pltpu.SemaphoreType.DMA,
            pltpu.SemaphoreType.DMA,
        ],
    )(_sc_body)
    return k(rssi_tm, bssid_tm, table, pos_f, cls_f)


def kernel(rssi, bssid, bssid_table, pos_table, cls_token):
    rssi_tm = rssi.T
    bssid_tm = bssid.astype(jnp.int32).T
    pos_f = pos_table.reshape(pos_table.shape[0] * E)
    cls_f = cls_token.reshape(E)
    out = _anchor2token(rssi_tm, bssid_tm, bssid_table, pos_f, cls_f)
    return out.reshape(OROW, B, E).transpose(1, 0, 2)
